# Initial kernel scaffold; baseline (speedup 1.0000x reference)
import jax
import jax.numpy as jnp
from jax.experimental import pallas as pl


def kernel(x, pos, edge_index, edge_attr, graph_idx, W_node, b_node, W_edge, b_edge, W1_0, b1_0, W2_0, b2_0, W1_1, b1_1, W2_1, b2_1):
    raise NotImplementedError("write your pallas kernel here")


# speedup vs baseline: 1.8987x; 1.8987x over previous
import jax
import jax.numpy as jnp
from jax import lax
from jax.experimental import pallas as pl
from jax.experimental.pallas import tpu as pltpu
from jax.experimental.pallas import tpu_sc as plsc

N = 10000
H = 128
E = 320000
NC, NS = 2, 16
NW = NC * NS
EPW = E // NW
C = 80
NCHUNK = EPW // C
HALF = 5120
APAD = 5248
RPS = APAD // NS
OPS = HALF // NS
RB = 2000
GRID = N // RB

_mesh = plsc.VectorSubcoreMesh(
    core_axis_name="c", subcore_axis_name="s", num_cores=NC, num_subcores=NS)


def _sc_edge_body(lo_hbm, ei, ej, a_hbm, b_hbm, u_out, deg_out,
                  idx_i, idx_j, idx_adj, arow, brow, ones_v, zdeg,
                  lo_v, sem_a, sem_b, u_sh, deg_sh):
  c = lax.axis_index("c")
  s = lax.axis_index("s")
  wid = c * NS + s

  pltpu.sync_copy(lo_hbm, lo_v)

  z = jnp.zeros((16,), jnp.float32)
  def zrow_body(r, _):
    for k in range(H // 16):
      arow[r, pl.ds(k * 16, 16)] = z
    return 0
  lax.fori_loop(0, C, zrow_body, 0)
  for t in range(4):
    pltpu.sync_copy(arow, u_sh.at[pl.ds(s * RPS + t * C, C)])
  pltpu.sync_copy(arow.at[pl.ds(0, RPS - 4 * C)],
                  u_sh.at[pl.ds(s * RPS + 4 * C, RPS - 4 * C)])
  def zdeg_body(r, _):
    zdeg[r, :] = z
    return 0
  lax.fori_loop(0, RPS // 4, zdeg_body, 0)
  for t in range(4):
    pltpu.sync_copy(zdeg, deg_sh.at[pl.ds(s * RPS + t * (RPS // 4), RPS // 4)])
  one = jnp.ones((16,), jnp.float32)
  def ones_body(r, _):
    ones_v[r, :] = one
    return 0
  lax.fori_loop(0, C, ones_body, 0)
  plsc.subcore_barrier()

  def chunk(g, _):
    base = wid * EPW + g * C
    pltpu.sync_copy(ei.at[pl.ds(base, C)], idx_i)
    pltpu.sync_copy(ej.at[pl.ds(base, C)], idx_j)
    cp_a = pltpu.async_copy(a_hbm.at[idx_i], arow, sem_a)
    cp_b = pltpu.async_copy(b_hbm.at[idx_j], brow, sem_b)

    lo = lo_v[:]
    for m in range(C // 16):
      sl = pl.ds(m * 16, 16)
      v = idx_i[sl] - lo
      oob = (v < 0) | (v >= HALF)
      idx_adj[sl] = jnp.where(oob, HALF, v)

    cp_a.wait()
    cp_b.wait()

    def rbody(r, _):
      for k in range(H // 16):
        sl = pl.ds(k * 16, 16)
        t = arow[r, sl] + brow[r, sl]
        arow[r, sl] = t / (1.0 + jnp.exp(-t))
      return 0
    lax.fori_loop(0, C, rbody, 0)

    pltpu.sync_copy(arow, u_sh.at[idx_adj], add=True)
    pltpu.sync_copy(ones_v, deg_sh.at[idx_adj], add=True)
    return 0

  lax.fori_loop(0, NCHUNK, chunk, 0)
  plsc.subcore_barrier()

  pltpu.sync_copy(u_sh.at[pl.ds(s * OPS, OPS)], u_out.at[c, pl.ds(s * OPS, OPS)])
  pltpu.sync_copy(deg_sh.at[pl.ds(s * OPS, OPS)],
                  deg_out.at[c, pl.ds(s * OPS, OPS)])


_sc_edge = pl.kernel(
    _sc_edge_body,
    out_type=(jax.ShapeDtypeStruct((NC, HALF, H), jnp.float32),
              jax.ShapeDtypeStruct((NC, HALF, 16), jnp.float32)),
    mesh=_mesh,
    scratch_types=[
        pltpu.VMEM((C,), jnp.int32),
        pltpu.VMEM((C,), jnp.int32),
        pltpu.VMEM((C,), jnp.int32),
        pltpu.VMEM((C, H), jnp.float32),
        pltpu.VMEM((C, H), jnp.float32),
        pltpu.VMEM((C, 16), jnp.float32),
        pltpu.VMEM((RPS // 4, 16), jnp.float32),
        pltpu.VMEM((16,), jnp.int32),
        pltpu.SemaphoreType.DMA,
        pltpu.SemaphoreType.DMA,
        pltpu.VMEM_SHARED((APAD, H), jnp.float32),
        pltpu.VMEM_SHARED((APAD, 16), jnp.float32),
    ],
)


def _tc0_body(x_ref, pos_ref, wn_ref, bn_ref, h_ref, pos_out_ref):
  xx = jnp.nan_to_num(x_ref[...])
  h = jnp.dot(xx, wn_ref[...], preferred_element_type=jnp.float32) + bn_ref[...]
  h_ref[...] = jnp.nan_to_num(h, nan=0.0)
  pos_out_ref[...] = jnp.clip(jnp.nan_to_num(pos_ref[...]), -20.0, 20.0)


def _tc_ab_body(h_ref, w1a_ref, w1b_ref, b1_ref, a_ref, b_ref):
  h = h_ref[...]
  a_ref[...] = jnp.dot(h, w1a_ref[...],
                       preferred_element_type=jnp.float32) + b1_ref[...]
  b_ref[...] = jnp.dot(h, w1b_ref[...], preferred_element_type=jnp.float32)


def _tc_upd_body(h_ref, u_ref, deg_ref, w2_ref, b2_ref, h_out_ref):
  u = u_ref[0] + u_ref[1]
  deg = (deg_ref[0] + deg_ref[1])[:, 0:1]
  agg = jnp.dot(u, w2_ref[...], preferred_element_type=jnp.float32)
  agg = agg + deg * b2_ref[...]
  h_out_ref[...] = jax.nn.silu(h_ref[...] + agg)


def _row_spec(width):
  return pl.BlockSpec((RB, width), lambda i: (i, 0))


def _full_spec(shape):
  nd = len(shape)
  return pl.BlockSpec(shape, lambda i: (0,) * nd)


def _pair_spec(width):
  return pl.BlockSpec((NC, RB, width), lambda i: (0, i, 0))


_f32 = jnp.float32


def kernel(x, pos, edge_index, edge_attr, graph_idx, W_node, b_node, W_edge,
           b_edge, W1_0, b1_0, W2_0, b2_0, W1_1, b1_1, W2_1, b2_1):
  del edge_attr, graph_idx, W_edge, b_edge
  ei = edge_index[0]
  ej = edge_index[1]
  lo0 = jnp.zeros((16,), jnp.int32)
  lo1 = jnp.full((16,), HALF, jnp.int32)

  h0, pos_out = pl.pallas_call(
      _tc0_body,
      grid=(GRID,),
      in_specs=[_row_spec(H), _row_spec(3), _full_spec((H, H)),
                _full_spec((1, H))],
      out_specs=[_row_spec(H), _row_spec(3)],
      out_shape=(jax.ShapeDtypeStruct((N, H), _f32),
                 jax.ShapeDtypeStruct((N, 3), _f32)),
  )(x, pos, W_node, b_node.reshape(1, H))

  w1a = jnp.stack([W1_0[:H], W1_1[:H]])
  w1b = jnp.stack([W1_0[H:], W1_1[H:]])
  b1s = jnp.stack([b1_0.reshape(1, H), b1_1.reshape(1, H)])
  w2s = jnp.stack([W2_0, W2_1])
  b2s = jnp.stack([b2_0.reshape(1, H), b2_1.reshape(1, H)])

  def layer_step(h, ws):
    w1a_l, w1b_l, b1_l, w2_l, b2_l = ws
    a, b = pl.pallas_call(
        _tc_ab_body,
        grid=(GRID,),
        in_specs=[_row_spec(H), _full_spec((H, H)), _full_spec((H, H)),
                  _full_spec((1, H))],
        out_specs=[_row_spec(H), _row_spec(H)],
        out_shape=(jax.ShapeDtypeStruct((N, H), _f32),
                   jax.ShapeDtypeStruct((N, H), _f32)),
    )(h, w1a_l, w1b_l, b1_l)

    u_lo, deg_lo = _sc_edge(lo0, ei, ej, a, b)
    u_hi, deg_hi = _sc_edge(lo1, ei, ej, a, b)
    u_acc = jnp.concatenate([u_lo, u_hi], axis=1)
    deg_acc = jnp.concatenate([deg_lo, deg_hi], axis=1)

    h_next = pl.pallas_call(
        _tc_upd_body,
        grid=(GRID,),
        in_specs=[_row_spec(H), _pair_spec(H), _pair_spec(16),
                  _full_spec((H, H)), _full_spec((1, H))],
        out_specs=_row_spec(H),
        out_shape=jax.ShapeDtypeStruct((N, H), _f32),
    )(h, u_acc, deg_acc, w2_l, b2_l)
    return h_next, 0

  h2, _ = lax.scan(layer_step, h0, (w1a, w1b, b1s, w2s, b2s))
  return (h2, pos_out)
